# baseline (speedup 1.0000x reference)
import functools

import jax
import jax.numpy as jnp
from jax import lax
from jax.experimental import pallas as pl
from jax.experimental.pallas import tpu as pltpu
from jax.experimental.pallas import tpu_sc as plsc

VOCAB = 1000000
CTX = 2048
D = 64

NUM_CORES = 2
NUM_SUBCORES = 16
NUM_WORKERS = NUM_CORES * NUM_SUBCORES
NBUF = 4
EDGE = (VOCAB // 128) * 128


def _sinusoidal_pe(context_size, embedding_size):
    positions = jnp.arange(context_size, dtype=jnp.float32)
    indices = jnp.arange(embedding_size // 2, dtype=jnp.float32)
    scaling_factor = 10000.0 ** (2.0 * indices / embedding_size)
    angles = positions[:, None] / scaling_factor
    pe = jnp.zeros((context_size, embedding_size), dtype=jnp.float32)
    pe = pe.at[:, 0::2].set(jnp.sin(angles))
    pe = pe.at[:, 1::2].set(jnp.cos(angles))
    return pe


def _make_sc_kernel(batch, tok_per_worker):
    mesh = plsc.VectorSubcoreMesh(
        core_axis_name="c",
        subcore_axis_name="s",
        num_cores=NUM_CORES,
        num_subcores=NUM_SUBCORES,
    )

    @functools.partial(
        pl.kernel,
        out_type=jax.ShapeDtypeStruct((batch, D), jnp.float32),
        mesh=mesh,
        scratch_types=[
            pltpu.VMEM((tok_per_worker + 16,), jnp.int32),
            pltpu.VMEM((tok_per_worker, D), jnp.float32),
            pltpu.VMEM((2 * NBUF + 1, D, 128), jnp.float32),
            pltpu.VMEM((2 * 2 * NBUF, D), jnp.float32),
            pltpu.SemaphoreType.DMA,
            pltpu.SemaphoreType.DMA,
            pltpu.SemaphoreType.DMA,
            pltpu.SemaphoreType.DMA,
        ],
        compiler_params=pltpu.CompilerParams(needs_layout_passes=False),
    )
    def body(
        idx_hbm, pe_hbm, tableT_hbm, tail_hbm, out_hbm,
        idx_s, pe_v, bufs, ring, sem_a, sem_b, sem_pe, sem_o,
    ):
        wid = lax.axis_index("s") * NUM_CORES + lax.axis_index("c")
        base = wid * tok_per_worker
        pe_base = lax.rem(base, CTX)
        pltpu.sync_copy(
            idx_hbm.at[pl.ds(base, tok_per_worker)],
            idx_s.at[pl.ds(0, tok_per_worker)],
        )
        pltpu.sync_copy(tail_hbm, bufs.at[2 * NBUF])
        pltpu.async_copy(
            pe_hbm.at[pl.ds(pe_base, tok_per_worker)], pe_v, sem_pe
        )
        iota16 = lax.iota(jnp.int32, 16)
        n_chunks = tok_per_worker // NBUF
        pair_rows = 2 * NBUF

        def fire(ch, gbase, gsem):
            vec = idx_s[pl.ds(ch * NBUF, 16)]
            for j in range(NBUF):
                off = jnp.minimum((vec[j] // 128) * 128, EDGE - 128)
                pltpu.async_copy(
                    tableT_hbm.at[:, pl.ds(off, 128)], bufs.at[gbase + j], gsem
                )

        def drain_extract(ch, gbase, gsem, slot):
            vec = idx_s[pl.ds(ch * NBUF, 16)]
            for j in range(NBUF):
                pltpu.make_async_copy(
                    tableT_hbm.at[:, pl.ds(0, 128)], bufs.at[gbase + j], gsem
                ).wait()
            for j in range(NBUF):
                edge = vec[j] >= EDGE
                off = jnp.minimum((vec[j] // 128) * 128, EDGE - 128)
                sel = jnp.full(
                    (16,), jnp.where(edge, 2 * NBUF, gbase + j), jnp.int32
                )
                col = jnp.where(edge, vec[j] - EDGE, vec[j] - off)
                voff = jnp.full((16,), col, jnp.int32)
                r = ch * NBUF + j
                rr = slot * pair_rows + gbase + j
                for c in range(D // 16):
                    sl = pl.ds(16 * c, 16)
                    g = plsc.load_gather(bufs, [sel, iota16 + (16 * c), voff])
                    ring[rr, sl] = g + pe_v[r, sl]

        fire(0, 0, sem_a)
        pltpu.make_async_copy(
            pe_hbm.at[pl.ds(0, tok_per_worker)], pe_v, sem_pe
        ).wait()

        def do_pair(it, _):
            ch_a = 2 * it
            slot = lax.rem(it, 2)

            @pl.when(it > 0)
            def _():
                pltpu.make_async_copy(
                    ring.at[pl.ds(0, pair_rows)],
                    out_hbm.at[pl.ds(base, pair_rows)],
                    sem_o,
                ).wait()

            fire(ch_a + 1, NBUF, sem_b)
            drain_extract(ch_a, 0, sem_a, slot)
            fire(jnp.minimum(ch_a + 2, n_chunks - 2), 0, sem_a)
            drain_extract(ch_a + 1, NBUF, sem_b, slot)
            pltpu.async_copy(
                ring.at[pl.ds(slot * pair_rows, pair_rows)],
                out_hbm.at[pl.ds(base + it * pair_rows, pair_rows)],
                sem_o,
            )
            return 0

        lax.fori_loop(0, n_chunks // 2, do_pair, 0)
        for j in range(NBUF):
            pltpu.make_async_copy(
                tableT_hbm.at[:, pl.ds(0, 128)], bufs.at[j], sem_a
            ).wait()
        pltpu.make_async_copy(
            ring.at[pl.ds(0, pair_rows)],
            out_hbm.at[pl.ds(base, pair_rows)],
            sem_o,
        ).wait()

    return body


TC_K = 16


def _make_tc_kernel(s_base, m_tokens):
    grid = (m_tokens // TC_K,)

    def body(idx_ref, *refs):
        tbl_blks = refs[:TC_K]
        pe_b = refs[TC_K]
        out_b = refs[TC_K + 1]
        g = pl.program_id(0)
        lane = lax.broadcasted_iota(jnp.int32, (D, 128), 1)
        for k in range(TC_K):
            v = idx_ref[s_base + g * TC_K + k]
            voff = lax.rem(v, 128)
            blk = tbl_blks[k][...]
            col = jnp.sum(jnp.where(lane == voff, blk, 0.0), axis=1)
            out_b[k, :] = col + pe_b[k, :]

    tbl_specs = [
        pl.BlockSpec(
            (D, 128),
            lambda g, idx_ref, k=k: (0, idx_ref[s_base + g * TC_K + k] // 128),
        )
        for k in range(TC_K)
    ]
    pe_spec = pl.BlockSpec(
        (TC_K, D),
        lambda g, idx_ref: ((s_base // TC_K + g) % (CTX // TC_K), 0),
    )
    out_spec = pl.BlockSpec((TC_K, D), lambda g, idx_ref: (g, 0))
    return pl.pallas_call(
        body,
        grid_spec=pltpu.PrefetchScalarGridSpec(
            num_scalar_prefetch=1,
            grid=grid,
            in_specs=[*tbl_specs, pe_spec],
            out_specs=out_spec,
        ),
        out_shape=jax.ShapeDtypeStruct((m_tokens, D), jnp.float32),
    )


SC_TOKENS = 4096


def kernel(x, table):
    pe = _sinusoidal_pe(CTX, D)
    idx = x.reshape(-1).astype(jnp.int32)
    batch = idx.shape[0]
    tableT = table.T
    tail = jnp.pad(tableT[:, EDGE:], ((0, 0), (0, 128 - (VOCAB - EDGE))))
    s = SC_TOKENS
    out_sc = _make_sc_kernel(s, s // NUM_WORKERS)(idx[:s], pe, tableT, tail)
    out_tc = _make_tc_kernel(s, batch - s)(idx, *([tableT] * TC_K), pe)
    out = jnp.concatenate([out_sc, out_tc], axis=0)
    return out.reshape(x.shape + (D,))


# speedup vs baseline: 1.3802x; 1.3802x over previous
import functools

import jax
import jax.numpy as jnp
from jax import lax
from jax.experimental import pallas as pl
from jax.experimental.pallas import tpu as pltpu
from jax.experimental.pallas import tpu_sc as plsc

VOCAB = 1000000
CTX = 2048
D = 64

NUM_CORES = 2
NUM_SUBCORES = 16
NUM_WORKERS = NUM_CORES * NUM_SUBCORES
NBUF = 4
EDGE = (VOCAB // 128) * 128


def _sinusoidal_pe(context_size, embedding_size):
    positions = jnp.arange(context_size, dtype=jnp.float32)
    indices = jnp.arange(embedding_size // 2, dtype=jnp.float32)
    scaling_factor = 10000.0 ** (2.0 * indices / embedding_size)
    angles = positions[:, None] / scaling_factor
    pe = jnp.zeros((context_size, embedding_size), dtype=jnp.float32)
    pe = pe.at[:, 0::2].set(jnp.sin(angles))
    pe = pe.at[:, 1::2].set(jnp.cos(angles))
    return pe


def _make_sc_kernel(batch, tok_per_worker):
    mesh = plsc.VectorSubcoreMesh(
        core_axis_name="c",
        subcore_axis_name="s",
        num_cores=NUM_CORES,
        num_subcores=NUM_SUBCORES,
    )

    @functools.partial(
        pl.kernel,
        out_type=jax.ShapeDtypeStruct((batch, D), jnp.float32),
        mesh=mesh,
        scratch_types=[
            pltpu.VMEM((tok_per_worker + 16,), jnp.int32),
            pltpu.VMEM((tok_per_worker, D), jnp.float32),
            pltpu.VMEM((2 * NBUF + 1, D, 128), jnp.float32),
            pltpu.VMEM((2 * 2 * NBUF, D), jnp.float32),
            pltpu.SemaphoreType.DMA,
            pltpu.SemaphoreType.DMA,
            pltpu.SemaphoreType.DMA,
            pltpu.SemaphoreType.DMA,
        ],
        compiler_params=pltpu.CompilerParams(needs_layout_passes=False),
    )
    def body(
        idx_hbm, pe_hbm, tableT_hbm, tail_hbm, out_hbm,
        idx_s, pe_v, bufs, ring, sem_a, sem_b, sem_pe, sem_o,
    ):
        wid = lax.axis_index("s") * NUM_CORES + lax.axis_index("c")
        base = wid * tok_per_worker
        pe_base = lax.rem(base, CTX)
        pltpu.sync_copy(
            idx_hbm.at[pl.ds(base, tok_per_worker)],
            idx_s.at[pl.ds(0, tok_per_worker)],
        )
        pltpu.sync_copy(tail_hbm, bufs.at[2 * NBUF])
        pltpu.async_copy(
            pe_hbm.at[pl.ds(pe_base, tok_per_worker)], pe_v, sem_pe
        )
        iota16 = lax.iota(jnp.int32, 16)
        n_chunks = tok_per_worker // NBUF
        pair_rows = 2 * NBUF

        def fire(ch, gbase, gsem):
            vec = idx_s[pl.ds(ch * NBUF, 16)]
            for j in range(NBUF):
                off = jnp.minimum((vec[j] // 128) * 128, EDGE - 128)
                pltpu.async_copy(
                    tableT_hbm.at[:, pl.ds(off, 128)], bufs.at[gbase + j], gsem
                )

        def drain_extract(ch, gbase, gsem, slot):
            vec = idx_s[pl.ds(ch * NBUF, 16)]
            for j in range(NBUF):
                pltpu.make_async_copy(
                    tableT_hbm.at[:, pl.ds(0, 128)], bufs.at[gbase + j], gsem
                ).wait()
            for j in range(NBUF):
                edge = vec[j] >= EDGE
                off = jnp.minimum((vec[j] // 128) * 128, EDGE - 128)
                sel = jnp.full(
                    (16,), jnp.where(edge, 2 * NBUF, gbase + j), jnp.int32
                )
                col = jnp.where(edge, vec[j] - EDGE, vec[j] - off)
                voff = jnp.full((16,), col, jnp.int32)
                r = ch * NBUF + j
                rr = slot * pair_rows + gbase + j
                for c in range(D // 16):
                    sl = pl.ds(16 * c, 16)
                    g = plsc.load_gather(bufs, [sel, iota16 + (16 * c), voff])
                    ring[rr, sl] = g + pe_v[r, sl]

        fire(0, 0, sem_a)
        pltpu.make_async_copy(
            pe_hbm.at[pl.ds(0, tok_per_worker)], pe_v, sem_pe
        ).wait()

        def do_pair(it, _):
            ch_a = 2 * it
            slot = lax.rem(it, 2)

            @pl.when(it > 0)
            def _():
                pltpu.make_async_copy(
                    ring.at[pl.ds(0, pair_rows)],
                    out_hbm.at[pl.ds(base, pair_rows)],
                    sem_o,
                ).wait()

            fire(ch_a + 1, NBUF, sem_b)
            drain_extract(ch_a, 0, sem_a, slot)
            fire(jnp.minimum(ch_a + 2, n_chunks - 2), 0, sem_a)
            drain_extract(ch_a + 1, NBUF, sem_b, slot)
            pltpu.async_copy(
                ring.at[pl.ds(slot * pair_rows, pair_rows)],
                out_hbm.at[pl.ds(base + it * pair_rows, pair_rows)],
                sem_o,
            )
            return 0

        lax.fori_loop(0, n_chunks // 2, do_pair, 0)
        for j in range(NBUF):
            pltpu.make_async_copy(
                tableT_hbm.at[:, pl.ds(0, 128)], bufs.at[j], sem_a
            ).wait()
        pltpu.make_async_copy(
            ring.at[pl.ds(0, pair_rows)],
            out_hbm.at[pl.ds(base, pair_rows)],
            sem_o,
        ).wait()

    return body


TC_K = 16


def _make_tc_kernel(s_base, m_tokens):
    grid = (m_tokens // TC_K,)

    def body(idx_ref, *refs):
        tbl_blks = refs[:TC_K]
        pe_b = refs[TC_K]
        out_b = refs[TC_K + 1]
        g = pl.program_id(0)
        lane = lax.broadcasted_iota(jnp.int32, (128, 1), 0)
        for k in range(TC_K):
            v = idx_ref[s_base + g * TC_K + k]
            voff = lax.rem(v, 128)
            onehot = (lane == voff).astype(jnp.float32)
            col = jnp.dot(
                tbl_blks[k][...], onehot, preferred_element_type=jnp.float32
            )
            out_b[k, :] = col[:, 0] + pe_b[k, :]

    tbl_specs = [
        pl.BlockSpec(
            (D, 128),
            lambda g, idx_ref, k=k: (0, idx_ref[s_base + g * TC_K + k] // 128),
        )
        for k in range(TC_K)
    ]
    pe_spec = pl.BlockSpec(
        (TC_K, D),
        lambda g, idx_ref: ((s_base // TC_K + g) % (CTX // TC_K), 0),
    )
    out_spec = pl.BlockSpec((TC_K, D), lambda g, idx_ref: (g, 0))
    return pl.pallas_call(
        body,
        grid_spec=pltpu.PrefetchScalarGridSpec(
            num_scalar_prefetch=1,
            grid=grid,
            in_specs=[*tbl_specs, pe_spec],
            out_specs=out_spec,
        ),
        out_shape=jax.ShapeDtypeStruct((m_tokens, D), jnp.float32),
    )


SC_TOKENS = 4096


def kernel(x, table):
    pe = _sinusoidal_pe(CTX, D)
    idx = x.reshape(-1).astype(jnp.int32)
    batch = idx.shape[0]
    tableT = table.T
    tail = jnp.pad(tableT[:, EDGE:], ((0, 0), (0, 128 - (VOCAB - EDGE))))
    s = SC_TOKENS
    out_sc = _make_sc_kernel(s, s // NUM_WORKERS)(idx[:s], pe, tableT, tail)
    out_tc = _make_tc_kernel(s, batch - s)(idx, *([tableT] * TC_K), pe)
    out = jnp.concatenate([out_sc, out_tc], axis=0)
    return out.reshape(x.shape + (D,))


# speedup vs baseline: 2.6382x; 1.9115x over previous
import functools

import numpy as np

import jax
import jax.numpy as jnp
from jax import lax
from jax.experimental import pallas as pl
from jax.experimental.pallas import tpu as pltpu
from jax.experimental.pallas import tpu_sc as plsc

VOCAB = 1000000
CTX = 2048
D = 64

NUM_CORES = 2
NUM_SUBCORES = 16
NUM_WORKERS = NUM_CORES * NUM_SUBCORES
NBUF = 4
EDGE = (VOCAB // 128) * 128


def _sinusoidal_pe(context_size, embedding_size):
    positions = np.arange(context_size, dtype=np.float32)
    indices = np.arange(embedding_size // 2, dtype=np.float32)
    scaling_factor = 10000.0 ** (2.0 * indices / embedding_size)
    angles = positions[:, None] / scaling_factor
    pe = np.zeros((context_size, embedding_size), dtype=np.float32)
    pe[:, 0::2] = np.sin(angles)
    pe[:, 1::2] = np.cos(angles)
    return jnp.asarray(pe)


def _make_sc_kernel(batch, tok_per_worker):
    mesh = plsc.VectorSubcoreMesh(
        core_axis_name="c",
        subcore_axis_name="s",
        num_cores=NUM_CORES,
        num_subcores=NUM_SUBCORES,
    )

    @functools.partial(
        pl.kernel,
        out_type=jax.ShapeDtypeStruct((batch, D), jnp.float32),
        mesh=mesh,
        scratch_types=[
            pltpu.VMEM((tok_per_worker + 16,), jnp.int32),
            pltpu.VMEM((tok_per_worker, D), jnp.float32),
            pltpu.VMEM((2 * NBUF + 1, D, 128), jnp.float32),
            pltpu.VMEM((2 * 2 * NBUF, D), jnp.float32),
            pltpu.SemaphoreType.DMA,
            pltpu.SemaphoreType.DMA,
            pltpu.SemaphoreType.DMA,
            pltpu.SemaphoreType.DMA,
        ],
        compiler_params=pltpu.CompilerParams(needs_layout_passes=False),
    )
    def body(
        idx_hbm, pe_hbm, tableT_hbm, tail_hbm, out_hbm,
        idx_s, pe_v, bufs, ring, sem_a, sem_b, sem_pe, sem_o,
    ):
        wid = lax.axis_index("s") * NUM_CORES + lax.axis_index("c")
        base = wid * tok_per_worker
        pe_base = lax.rem(base, CTX)
        pltpu.sync_copy(
            idx_hbm.at[pl.ds(base, tok_per_worker)],
            idx_s.at[pl.ds(0, tok_per_worker)],
        )
        pltpu.sync_copy(tail_hbm, bufs.at[2 * NBUF])
        pltpu.async_copy(
            pe_hbm.at[pl.ds(pe_base, tok_per_worker)], pe_v, sem_pe
        )
        iota16 = lax.iota(jnp.int32, 16)
        n_chunks = tok_per_worker // NBUF
        pair_rows = 2 * NBUF

        def fire(ch, gbase, gsem):
            vec = idx_s[pl.ds(ch * NBUF, 16)]
            for j in range(NBUF):
                off = jnp.minimum((vec[j] // 128) * 128, EDGE - 128)
                pltpu.async_copy(
                    tableT_hbm.at[:, pl.ds(off, 128)], bufs.at[gbase + j], gsem
                )

        def drain_extract(ch, gbase, gsem, slot):
            vec = idx_s[pl.ds(ch * NBUF, 16)]
            for j in range(NBUF):
                pltpu.make_async_copy(
                    tableT_hbm.at[:, pl.ds(0, 128)], bufs.at[gbase + j], gsem
                ).wait()
            for j in range(NBUF):
                edge = vec[j] >= EDGE
                off = jnp.minimum((vec[j] // 128) * 128, EDGE - 128)
                sel = jnp.full(
                    (16,), jnp.where(edge, 2 * NBUF, gbase + j), jnp.int32
                )
                col = jnp.where(edge, vec[j] - EDGE, vec[j] - off)
                voff = jnp.full((16,), col, jnp.int32)
                r = ch * NBUF + j
                rr = slot * pair_rows + gbase + j
                for c in range(D // 16):
                    sl = pl.ds(16 * c, 16)
                    g = plsc.load_gather(bufs, [sel, iota16 + (16 * c), voff])
                    ring[rr, sl] = g + pe_v[r, sl]

        fire(0, 0, sem_a)
        pltpu.make_async_copy(
            pe_hbm.at[pl.ds(0, tok_per_worker)], pe_v, sem_pe
        ).wait()

        def do_pair(it, _):
            ch_a = 2 * it
            slot = lax.rem(it, 2)

            @pl.when(it > 0)
            def _():
                pltpu.make_async_copy(
                    ring.at[pl.ds(0, pair_rows)],
                    out_hbm.at[pl.ds(base, pair_rows)],
                    sem_o,
                ).wait()

            fire(ch_a + 1, NBUF, sem_b)
            drain_extract(ch_a, 0, sem_a, slot)
            fire(jnp.minimum(ch_a + 2, n_chunks - 2), 0, sem_a)
            drain_extract(ch_a + 1, NBUF, sem_b, slot)
            pltpu.async_copy(
                ring.at[pl.ds(slot * pair_rows, pair_rows)],
                out_hbm.at[pl.ds(base + it * pair_rows, pair_rows)],
                sem_o,
            )
            return 0

        lax.fori_loop(0, n_chunks // 2, do_pair, 0)
        for j in range(NBUF):
            pltpu.make_async_copy(
                tableT_hbm.at[:, pl.ds(0, 128)], bufs.at[j], sem_a
            ).wait()
        pltpu.make_async_copy(
            ring.at[pl.ds(0, pair_rows)],
            out_hbm.at[pl.ds(base, pair_rows)],
            sem_o,
        ).wait()

    return body


def kernel(x, table):
    pe = _sinusoidal_pe(CTX, D)
    idx = x.reshape(-1).astype(jnp.int32)
    batch = idx.shape[0]
    tok_per_worker = batch // NUM_WORKERS
    tableT = table.T
    tail = jnp.pad(tableT[:, EDGE:], ((0, 0), (0, 128 - (VOCAB - EDGE))))
    out = _make_sc_kernel(batch, tok_per_worker)(idx, pe, tableT, tail)
    return out.reshape(x.shape + (D,))


# speedup vs baseline: 2.6473x; 1.0035x over previous
import functools

import numpy as np

import jax
import jax.numpy as jnp
from jax import lax
from jax.experimental import pallas as pl
from jax.experimental.pallas import tpu as pltpu
from jax.experimental.pallas import tpu_sc as plsc

VOCAB = 1000000
CTX = 2048
D = 64

NUM_CORES = 2
NUM_SUBCORES = 16
NUM_WORKERS = NUM_CORES * NUM_SUBCORES
NBUF = 4
EDGE = (VOCAB // 128) * 128


def _sinusoidal_pe(context_size, embedding_size):
    positions = np.arange(context_size, dtype=np.float32)
    indices = np.arange(embedding_size // 2, dtype=np.float32)
    scaling_factor = 10000.0 ** (2.0 * indices / embedding_size)
    angles = positions[:, None] / scaling_factor
    pe = np.zeros((context_size, embedding_size), dtype=np.float32)
    pe[:, 0::2] = np.sin(angles)
    pe[:, 1::2] = np.cos(angles)
    return jnp.asarray(pe)


def _make_sc_kernel(batch, tok_per_worker):
    mesh = plsc.VectorSubcoreMesh(
        core_axis_name="c",
        subcore_axis_name="s",
        num_cores=NUM_CORES,
        num_subcores=NUM_SUBCORES,
    )

    @functools.partial(
        pl.kernel,
        out_type=jax.ShapeDtypeStruct((batch, D), jnp.float32),
        mesh=mesh,
        scratch_types=[
            pltpu.VMEM((tok_per_worker + 16,), jnp.int32),
            pltpu.VMEM((tok_per_worker, D), jnp.float32),
            pltpu.VMEM((2 * NBUF + 1, D, 128), jnp.float32),
            pltpu.VMEM((2 * 2 * NBUF, D), jnp.float32),
            pltpu.SemaphoreType.DMA,
            pltpu.SemaphoreType.DMA,
            pltpu.SemaphoreType.DMA,
            pltpu.SemaphoreType.DMA,
        ],
        compiler_params=pltpu.CompilerParams(needs_layout_passes=False),
    )
    def body(
        idx_hbm, pe_hbm, tableT_hbm, tail_hbm, out_hbm,
        idx_s, pe_v, bufs, ring, sem_a, sem_b, sem_pe, sem_o,
    ):
        wid = lax.axis_index("s") * NUM_CORES + lax.axis_index("c")
        base = wid * tok_per_worker
        pe_base = lax.rem(base, CTX)
        pltpu.sync_copy(
            idx_hbm.at[pl.ds(base, tok_per_worker)],
            idx_s.at[pl.ds(0, tok_per_worker)],
        )
        pltpu.sync_copy(tail_hbm, bufs.at[2 * NBUF])
        pltpu.async_copy(
            pe_hbm.at[pl.ds(pe_base, tok_per_worker)], pe_v, sem_pe
        )
        iota16 = lax.iota(jnp.int32, 16)
        n_chunks = tok_per_worker // NBUF
        pair_rows = 2 * NBUF

        def fire(ch, gbase, gsem):
            vec = idx_s[pl.ds(ch * NBUF, 16)]
            for j in range(NBUF):
                off = jnp.minimum((vec[j] // 128) * 128, EDGE - 128)
                for h in range(2):
                    pltpu.async_copy(
                        tableT_hbm.at[pl.ds(32 * h, 32), pl.ds(off, 128)],
                        bufs.at[gbase + j, pl.ds(32 * h, 32)],
                        gsem,
                    )

        def drain_extract(ch, gbase, gsem, slot):
            vec = idx_s[pl.ds(ch * NBUF, 16)]
            for j in range(NBUF):
                for h in range(2):
                    pltpu.make_async_copy(
                        tableT_hbm.at[pl.ds(0, 32), pl.ds(0, 128)],
                        bufs.at[gbase + j, pl.ds(32 * h, 32)],
                        gsem,
                    ).wait()
            for j in range(NBUF):
                edge = vec[j] >= EDGE
                off = jnp.minimum((vec[j] // 128) * 128, EDGE - 128)
                sel = jnp.full(
                    (16,), jnp.where(edge, 2 * NBUF, gbase + j), jnp.int32
                )
                col = jnp.where(edge, vec[j] - EDGE, vec[j] - off)
                voff = jnp.full((16,), col, jnp.int32)
                r = ch * NBUF + j
                rr = slot * pair_rows + gbase + j
                for c in range(D // 16):
                    sl = pl.ds(16 * c, 16)
                    g = plsc.load_gather(bufs, [sel, iota16 + (16 * c), voff])
                    ring[rr, sl] = g + pe_v[r, sl]

        fire(0, 0, sem_a)
        pltpu.make_async_copy(
            pe_hbm.at[pl.ds(0, tok_per_worker)], pe_v, sem_pe
        ).wait()

        def do_pair(it, _):
            ch_a = 2 * it
            slot = lax.rem(it, 2)

            @pl.when(it > 0)
            def _():
                pltpu.make_async_copy(
                    ring.at[pl.ds(0, pair_rows)],
                    out_hbm.at[pl.ds(base, pair_rows)],
                    sem_o,
                ).wait()

            fire(ch_a + 1, NBUF, sem_b)
            drain_extract(ch_a, 0, sem_a, slot)
            fire(jnp.minimum(ch_a + 2, n_chunks - 2), 0, sem_a)
            drain_extract(ch_a + 1, NBUF, sem_b, slot)
            pltpu.async_copy(
                ring.at[pl.ds(slot * pair_rows, pair_rows)],
                out_hbm.at[pl.ds(base + it * pair_rows, pair_rows)],
                sem_o,
            )
            return 0

        lax.fori_loop(0, n_chunks // 2, do_pair, 0)
        for j in range(NBUF):
            pltpu.make_async_copy(
                tableT_hbm.at[:, pl.ds(0, 128)], bufs.at[j], sem_a
            ).wait()
        pltpu.make_async_copy(
            ring.at[pl.ds(0, pair_rows)],
            out_hbm.at[pl.ds(base, pair_rows)],
            sem_o,
        ).wait()

    return body


def kernel(x, table):
    pe = _sinusoidal_pe(CTX, D)
    idx = x.reshape(-1).astype(jnp.int32)
    batch = idx.shape[0]
    tok_per_worker = batch // NUM_WORKERS
    tableT = table.T
    tail = jnp.pad(tableT[:, EDGE:], ((0, 0), (0, 128 - (VOCAB - EDGE))))
    out = _make_sc_kernel(batch, tok_per_worker)(idx, pe, tableT, tail)
    return out.reshape(x.shape + (D,))
